# baseline (speedup 1.0000x reference)
import functools

import jax
import jax.numpy as jnp
from jax import lax
from jax.experimental import pallas as pl
from jax.experimental.pallas import tpu as pltpu
from jax.experimental.pallas import tpu_sc as plsc

NC = 2
NS = 16
NW = NC * NS
LANES = 16
CHUNK = 128
NBUF = 3
UNROLL = 4


def _make_kernel(N, V, L, D):
    rows_per_w = N // NW
    n_chunks = rows_per_w // CHUNK
    mesh = plsc.VectorSubcoreMesh(
        core_axis_name="c", subcore_axis_name="s", num_cores=NC, num_subcores=NS
    )

    @functools.partial(
        pl.kernel,
        out_type=jax.ShapeDtypeStruct((N, D), jnp.float32),
        mesh=mesh,
        scratch_types=[
            pltpu.VMEM((L + CHUNK, D), jnp.float32),
            pltpu.VMEM((n_chunks, CHUNK), jnp.int32),
            pltpu.VMEM((NBUF, CHUNK, D), jnp.float32),
            pltpu.SemaphoreType.DMA((NBUF,)),
            pltpu.SemaphoreType.DMA((NBUF,)),
        ],
    )
    def k(x2_hbm, tok_hbm, pos_hbm, out_hbm, pos2_v, idxs_v, rows_v, semg, sems):
        wid = lax.axis_index("s") * NC + lax.axis_index("c")
        base = wid * rows_per_w
        pltpu.sync_copy(pos_hbm, pos2_v.at[pl.ds(0, L)])
        pltpu.sync_copy(pos_hbm.at[pl.ds(0, CHUNK)], pos2_v.at[pl.ds(L, CHUNK)])
        pltpu.sync_copy(x2_hbm.at[pl.ds(wid * n_chunks, n_chunks)], idxs_v)
        for c in range(NBUF - 1):
            pltpu.async_copy(tok_hbm.at[idxs_v.at[c]], rows_v.at[c], semg.at[c])

        def chunk_body(c, carry):
            p = lax.rem(c, NBUF)

            pltpu.make_async_copy(
                tok_hbm.at[pl.ds(0, CHUNK)], rows_v.at[p], semg.at[p]
            ).wait()

            p0 = lax.rem(c * CHUNK, L)
            nj = D // LANES

            def row_body(i, _):
                r = i * UNROLL
                pv = [
                    pos2_v[p0 + r + u, pl.ds(j * LANES, LANES)]
                    for u in range(UNROLL)
                    for j in range(nj)
                ]
                for u in range(UNROLL):
                    for j in range(nj):
                        plsc.addupdate(
                            rows_v.at[p, r + u, pl.ds(j * LANES, LANES)],
                            pv[u * nj + j],
                        )
                return _

            lax.fori_loop(0, 1, row_body, 0)
            H = CHUNK // 2
            pltpu.async_copy(
                rows_v.at[p, pl.ds(0, H)],
                out_hbm.at[pl.ds(base + c * CHUNK, H)],
                sems.at[p],
            )
            pltpu.async_copy(
                rows_v.at[p, pl.ds(H, H)],
                out_hbm.at[pl.ds(base + c * CHUNK + H, H)],
                sems.at[p],
            )

            pnext = lax.rem(c + NBUF - 1, NBUF)

            @pl.when(c + NBUF - 1 < n_chunks)
            def _prefetch():
                @pl.when(c >= 1)
                def _drain_store():
                    pltpu.make_async_copy(
                        rows_v.at[pnext],
                        out_hbm.at[pl.ds(base, CHUNK)],
                        sems.at[pnext],
                    ).wait()

                pltpu.async_copy(
                    tok_hbm.at[idxs_v.at[c + NBUF - 1, pl.ds(0, H)]],
                    rows_v.at[pnext, pl.ds(0, H)],
                    semg.at[pnext],
                )
                pltpu.async_copy(
                    tok_hbm.at[idxs_v.at[c + NBUF - 1, pl.ds(H, H)]],
                    rows_v.at[pnext, pl.ds(H, H)],
                    semg.at[pnext],
                )
            return carry

        lax.fori_loop(0, n_chunks, chunk_body, 0)
        for p in range(NBUF):
            pltpu.make_async_copy(
                rows_v.at[p], out_hbm.at[pl.ds(base, CHUNK)], sems.at[p]
            ).wait()

    return k


def kernel(x, token_table, pos_table):
    B, L = x.shape
    V, D = token_table.shape
    N = B * L
    x2 = x.reshape(N // CHUNK, CHUNK).astype(jnp.int32)
    out = _make_kernel(N, V, L, D)(x2, token_table, pos_table)
    return out.reshape(B, L, D)


# speedup vs baseline: 1.0102x; 1.0102x over previous
import functools

import jax
import jax.numpy as jnp
from jax import lax
from jax.experimental import pallas as pl
from jax.experimental.pallas import tpu as pltpu
from jax.experimental.pallas import tpu_sc as plsc

NC = 2
NS = 16
NW = NC * NS
LANES = 16
CHUNK = 128
NBUF = 4
RAMP = 336


def _make_kernel(N, V, L, D):
    rows_per_w = N // NW
    n_chunks = rows_per_w // CHUNK
    mesh = plsc.VectorSubcoreMesh(
        core_axis_name="c", subcore_axis_name="s", num_cores=NC, num_subcores=NS
    )

    @functools.partial(
        pl.kernel,
        out_type=jax.ShapeDtypeStruct((N, D), jnp.float32),
        mesh=mesh,
        scratch_types=[
            pltpu.VMEM_SHARED((L + CHUNK, D), jnp.float32),
            pltpu.VMEM((n_chunks, CHUNK), jnp.int32),
            pltpu.VMEM((RAMP,), jnp.int32),
            pltpu.VMEM((NBUF, CHUNK, D), jnp.float32),
            pltpu.SemaphoreType.DMA((NBUF,)),
            pltpu.SemaphoreType.DMA((NBUF,)),
            pltpu.SemaphoreType.DMA((NBUF,)),
        ],
    )
    def k(x2_hbm, tok_hbm, pos_hbm, out_hbm, pos_sp, idxs_v, ramp_v, rows_v,
          semg, sema, sems):
        cid = lax.axis_index("c")
        sid = lax.axis_index("s")
        wid = sid * NC + cid
        base = wid * rows_per_w

        @pl.when(sid == 0)
        def _stage_pos():
            pltpu.sync_copy(pos_hbm, pos_sp.at[pl.ds(0, L)])
            pltpu.sync_copy(pos_hbm.at[pl.ds(0, CHUNK)], pos_sp.at[pl.ds(L, CHUNK)])

        pltpu.sync_copy(x2_hbm.at[pl.ds(wid * n_chunks, n_chunks)], idxs_v)
        iota = lax.iota(jnp.int32, LANES)
        for g in range(RAMP // LANES):
            ramp_v[pl.ds(g * LANES, LANES)] = iota + (g * LANES)
        plsc.subcore_barrier()

        for c in range(2):
            pltpu.async_copy(tok_hbm.at[idxs_v.at[c]], rows_v.at[c], semg.at[c])

        def chunk_body(c, carry):
            p = lax.rem(c, NBUF)

            pltpu.make_async_copy(
                tok_hbm.at[pl.ds(0, CHUNK)], rows_v.at[p], semg.at[p]
            ).wait()
            p0 = lax.rem(c * CHUNK, L)
            pltpu.async_copy(
                pos_sp.at[ramp_v.at[pl.ds(p0, CHUNK)]],
                rows_v.at[p],
                sema.at[p],
                add=True,
            )

            @pl.when(c >= 1)
            def _store_prev():
                pm = lax.rem(c + NBUF - 1, NBUF)
                pltpu.make_async_copy(
                    pos_sp.at[pl.ds(0, CHUNK)], rows_v.at[pm], sema.at[pm]
                ).wait()
                pltpu.async_copy(
                    rows_v.at[pm],
                    out_hbm.at[pl.ds(base + (c - 1) * CHUNK, CHUNK)],
                    sems.at[pm],
                )

            @pl.when(c + 2 < n_chunks)
            def _prefetch():
                pnext = lax.rem(c + 2, NBUF)

                @pl.when(c >= 2)
                def _drain_store():
                    pltpu.make_async_copy(
                        rows_v.at[pnext],
                        out_hbm.at[pl.ds(base, CHUNK)],
                        sems.at[pnext],
                    ).wait()

                pltpu.async_copy(
                    tok_hbm.at[idxs_v.at[c + 2]],
                    rows_v.at[pnext],
                    semg.at[pnext],
                )
            return carry

        lax.fori_loop(0, n_chunks, chunk_body, 0)

        plast = lax.rem(n_chunks - 1, NBUF)
        pltpu.make_async_copy(
            pos_sp.at[pl.ds(0, CHUNK)], rows_v.at[plast], sema.at[plast]
        ).wait()
        pltpu.async_copy(
            rows_v.at[plast],
            out_hbm.at[pl.ds(base + (n_chunks - 1) * CHUNK, CHUNK)],
            sems.at[plast],
        )
        for p in range(NBUF):
            pltpu.make_async_copy(
                rows_v.at[p], out_hbm.at[pl.ds(base, CHUNK)], sems.at[p]
            ).wait()

    return k


def kernel(x, token_table, pos_table):
    B, L = x.shape
    V, D = token_table.shape
    N = B * L
    x2 = x.reshape(N // CHUNK, CHUNK).astype(jnp.int32)
    out = _make_kernel(N, V, L, D)(x2, token_table, pos_table)
    return out.reshape(B, L, D)


# speedup vs baseline: 1.0123x; 1.0021x over previous
import functools

import jax
import jax.numpy as jnp
from jax import lax
from jax.experimental import pallas as pl
from jax.experimental.pallas import tpu as pltpu
from jax.experimental.pallas import tpu_sc as plsc

NC = 2
NS = 16
NW = NC * NS
LANES = 16
CHUNK = 128
NBUF = 5
DEPTH = 3
RAMP = 336


def _make_kernel(N, V, L, D):
    rows_per_w = N // NW
    n_chunks = rows_per_w // CHUNK
    mesh = plsc.VectorSubcoreMesh(
        core_axis_name="c", subcore_axis_name="s", num_cores=NC, num_subcores=NS
    )

    @functools.partial(
        pl.kernel,
        out_type=jax.ShapeDtypeStruct((N, D), jnp.float32),
        mesh=mesh,
        scratch_types=[
            pltpu.VMEM_SHARED((L + CHUNK, D), jnp.float32),
            pltpu.VMEM((n_chunks, CHUNK), jnp.int32),
            pltpu.VMEM((RAMP,), jnp.int32),
            pltpu.VMEM((NBUF, CHUNK, D), jnp.float32),
            pltpu.SemaphoreType.DMA((NBUF,)),
            pltpu.SemaphoreType.DMA((NBUF,)),
            pltpu.SemaphoreType.DMA((NBUF,)),
        ],
    )
    def k(x2_hbm, tok_hbm, pos_hbm, out_hbm, pos_sp, idxs_v, ramp_v, rows_v,
          semg, sema, sems):
        cid = lax.axis_index("c")
        sid = lax.axis_index("s")
        wid = sid * NC + cid
        base = wid * rows_per_w

        @pl.when(sid == 0)
        def _stage_pos():
            pltpu.sync_copy(pos_hbm, pos_sp.at[pl.ds(0, L)])
            pltpu.sync_copy(pos_hbm.at[pl.ds(0, CHUNK)], pos_sp.at[pl.ds(L, CHUNK)])

        pltpu.sync_copy(x2_hbm.at[pl.ds(wid * n_chunks, n_chunks)], idxs_v)
        iota = lax.iota(jnp.int32, LANES)
        for g in range(RAMP // LANES):
            ramp_v[pl.ds(g * LANES, LANES)] = iota + (g * LANES)
        plsc.subcore_barrier()

        for c in range(DEPTH):
            pltpu.async_copy(tok_hbm.at[idxs_v.at[c]], rows_v.at[c], semg.at[c])

        def chunk_body(c, carry):
            p = lax.rem(c, NBUF)

            pltpu.make_async_copy(
                tok_hbm.at[pl.ds(0, CHUNK)], rows_v.at[p], semg.at[p]
            ).wait()
            p0 = lax.rem(c * CHUNK, L)
            pltpu.async_copy(
                pos_sp.at[ramp_v.at[pl.ds(p0, CHUNK)]],
                rows_v.at[p],
                sema.at[p],
                add=True,
            )

            @pl.when(c >= 1)
            def _store_prev():
                pm = lax.rem(c + NBUF - 1, NBUF)
                pltpu.make_async_copy(
                    pos_sp.at[pl.ds(0, CHUNK)], rows_v.at[pm], sema.at[pm]
                ).wait()
                pltpu.async_copy(
                    rows_v.at[pm],
                    out_hbm.at[pl.ds(base + (c - 1) * CHUNK, CHUNK)],
                    sems.at[pm],
                )

            @pl.when(c + DEPTH < n_chunks)
            def _prefetch():
                pnext = lax.rem(c + DEPTH, NBUF)

                @pl.when(c >= NBUF - DEPTH)
                def _drain_store():
                    pltpu.make_async_copy(
                        rows_v.at[pnext],
                        out_hbm.at[pl.ds(base, CHUNK)],
                        sems.at[pnext],
                    ).wait()

                pltpu.async_copy(
                    tok_hbm.at[idxs_v.at[c + DEPTH]],
                    rows_v.at[pnext],
                    semg.at[pnext],
                )
            return carry

        lax.fori_loop(0, n_chunks, chunk_body, 0)

        plast = lax.rem(n_chunks - 1, NBUF)
        pltpu.make_async_copy(
            pos_sp.at[pl.ds(0, CHUNK)], rows_v.at[plast], sema.at[plast]
        ).wait()
        pltpu.async_copy(
            rows_v.at[plast],
            out_hbm.at[pl.ds(base + (n_chunks - 1) * CHUNK, CHUNK)],
            sems.at[plast],
        )
        for p in range(NBUF):
            pltpu.make_async_copy(
                rows_v.at[p], out_hbm.at[pl.ds(base, CHUNK)], sems.at[p]
            ).wait()

    return k


def kernel(x, token_table, pos_table):
    B, L = x.shape
    V, D = token_table.shape
    N = B * L
    x2 = x.reshape(N // CHUNK, CHUNK).astype(jnp.int32)
    out = _make_kernel(N, V, L, D)(x2, token_table, pos_table)
    return out.reshape(B, L, D)
